# baseline (speedup 1.0000x reference)
import jax
import jax.numpy as jnp
from jax.experimental import pallas as pl

NEG = -100000000.0
EPS = 1.1920929e-07
BIGI = 2**30


def _heads_kernel(vml_ref, vmm_ref, pmp_ref, pmm_ref, out_ref):
    br = vml_ref.shape[0]
    vml = vml_ref[...]
    vmm_f = vmm_ref[...].astype(jnp.float32)
    x = vml * (1.0 - vmm_f) + NEG * vmm_f
    m = jnp.max(x, axis=1, keepdims=True)
    e = jnp.exp(x - m)
    s = jnp.sum(e, axis=1, keepdims=True)
    sxe = jnp.sum(e * x, axis=1, keepdims=True)
    logs = jnp.log(s)
    lse = m + logs
    vm_lp = -logs[:, 0]
    vm_ent = jnp.where(m[:, 0] == NEG, 0.0, lse[:, 0] - sxe[:, 0] / s[:, 0])
    ii = jax.lax.broadcasted_iota(jnp.int32, x.shape, 1)
    sel_vm = jnp.min(jnp.where(x == m, ii, BIGI), axis=1)

    pp = pmp_ref[...]
    un = 1.0 - pmm_ref[...].astype(jnp.float32)
    p = pp * un
    S = jnp.sum(p, axis=1, keepdims=True)
    cnt = jnp.sum(un, axis=1, keepdims=True)
    small = S < 0.0001
    p2 = jnp.where(small, un, p)
    S2 = jnp.where(small, cnt, S)
    q = p2 / S2
    lq = jnp.log(jnp.clip(q, EPS, 1.0 - EPS))
    pm_ent = -jnp.sum(lq * q, axis=1)
    mq = jnp.max(q, axis=1, keepdims=True)
    jj = jax.lax.broadcasted_iota(jnp.int32, q.shape, 1)
    sel_pm = jnp.min(jnp.where(q == mq, jj, BIGI), axis=1)
    pm_lp = jnp.log(jnp.clip(mq[:, 0], EPS, 1.0 - EPS))

    lp = (vm_lp + pm_lp).view(jnp.int32)
    ent = (vm_ent + pm_ent).view(jnp.int32)
    out_ref[0, 0, :] = jnp.concatenate([sel_vm, sel_pm, lp, ent])


def kernel(vm_logits, vm_mask, pm_probs, pm_mask):
    B = vm_logits.shape[0]
    NV = vm_logits.shape[1]
    NP = pm_probs.shape[1]
    G = 1
    BR = B // G
    vm_mask = vm_mask.view(jnp.int8)
    pm_mask = pm_mask.view(jnp.int8)
    out = pl.pallas_call(
        _heads_kernel,
        grid=(G,),
        in_specs=[
            pl.BlockSpec((BR, NV), lambda i: (i, 0)),
            pl.BlockSpec((BR, NV), lambda i: (i, 0)),
            pl.BlockSpec((BR, NP), lambda i: (i, 0)),
            pl.BlockSpec((BR, NP), lambda i: (i, 0)),
        ],
        out_specs=pl.BlockSpec((1, 1, 4 * BR), lambda i: (i, 0, 0)),
        out_shape=jax.ShapeDtypeStruct((G, 1, 4 * BR), jnp.int32),
    )(vm_logits, vm_mask, pm_probs, pm_mask)
    out = out.reshape(G, 4, BR)
    sel_vm = out[:, 0, :].reshape(B)
    sel_pm = out[:, 1, :].reshape(B)
    lp = out[:, 2, :].reshape(B).view(jnp.float32)
    ent = out[:, 3, :].reshape(B).view(jnp.float32)
    return (sel_vm, sel_pm, lp, ent)


# speedup vs baseline: 1.1067x; 1.1067x over previous
import jax
import jax.numpy as jnp
from jax.experimental import pallas as pl

NEG = -100000000.0
EPS = 1.1920929e-07
BIGI = 2**30


def _heads_kernel(x_ref, p_ref, selvm_ref, selpm_ref, lp_ref, ent_ref):
    x = x_ref[...]
    m = jnp.max(x, axis=1, keepdims=True)
    e = jnp.exp(x - m)
    s = jnp.sum(e, axis=1, keepdims=True)
    sxe = jnp.sum(e * x, axis=1, keepdims=True)
    logs = jnp.log(s)
    lse = m + logs
    vm_lp = -logs[:, 0]
    vm_ent = jnp.where(m[:, 0] == NEG, 0.0, lse[:, 0] - sxe[:, 0] / s[:, 0])
    ii = jax.lax.broadcasted_iota(jnp.int32, x.shape, 1)
    sel_vm = jnp.min(jnp.where(x == m, ii, BIGI), axis=1)

    p = p_ref[...]
    S = jnp.sum(p, axis=1, keepdims=True)
    un = jnp.sign(p)
    cnt = jnp.sum(un, axis=1, keepdims=True)
    small = S < 0.0001
    p2 = jnp.where(small, un, p)
    S2 = jnp.where(small, cnt, S)
    q = p2 / S2
    lq = jnp.log(jnp.clip(q, EPS, 1.0 - EPS))
    pm_ent = -jnp.sum(lq * q, axis=1)
    mq = jnp.max(q, axis=1, keepdims=True)
    jj = jax.lax.broadcasted_iota(jnp.int32, q.shape, 1)
    sel_pm = jnp.min(jnp.where(q == mq, jj, BIGI), axis=1)
    pm_lp = jnp.log(jnp.clip(mq[:, 0], EPS, 1.0 - EPS))

    selvm_ref[...] = sel_vm
    selpm_ref[...] = sel_pm
    lp_ref[...] = vm_lp + pm_lp
    ent_ref[...] = vm_ent + pm_ent


def kernel(vm_logits, vm_mask, pm_probs, pm_mask):
    B = vm_logits.shape[0]
    x = jnp.where(vm_mask, NEG, vm_logits)
    p = jnp.where(pm_mask, 0.0, pm_probs)
    out = pl.pallas_call(
        _heads_kernel,
        out_shape=(
            jax.ShapeDtypeStruct((B,), jnp.int32),
            jax.ShapeDtypeStruct((B,), jnp.int32),
            jax.ShapeDtypeStruct((B,), jnp.float32),
            jax.ShapeDtypeStruct((B,), jnp.float32),
        ),
    )(x, p)
    return out
